# baseline (speedup 1.0000x reference)
import functools

import jax
import jax.numpy as jnp
from jax import lax
from jax.experimental import pallas as pl
from jax.experimental.pallas import tpu as pltpu
from jax.experimental.pallas import tpu_sc as plsc

_VOCAB = 50277
_D = 1024
_B = 4
_S = 4096
_N = _B * _S

_NC = 2
_NS = 16
_NW = _NC * _NS
_BPW = _N // _NW
_CH = 16
_NCHUNK = _BPW // _CH
_NBUF = 6


_WPR = _S // _BPW


def _gather_body(table_hbm, idx_hbm, out_hbm, idx_v, *scr):
    wid = lax.axis_index("s") * _NC + lax.axis_index("c")
    base = wid * _BPW
    pltpu.sync_copy(
        idx_hbm.at[wid // _WPR, pl.ds((wid % _WPR) * _BPW, _BPW)], idx_v)
    bufs = scr[:_NBUF]
    gsems = scr[_NBUF:2 * _NBUF]
    osems = scr[2 * _NBUF:3 * _NBUF]

    def gather(c):
        b = c % _NBUF
        return pltpu.async_copy(
            table_hbm.at[idx_v.at[pl.ds(c * _CH, _CH)]], bufs[b], gsems[b])

    def put(c):
        b = c % _NBUF
        return pltpu.async_copy(
            bufs[b], out_hbm.at[pl.ds(base + c * _CH, _CH)], osems[b])

    g = [None] * _NCHUNK
    o = [None] * _NCHUNK
    waited = set()
    for c in range(_NBUF - 1):
        g[c] = gather(c)
    for c in range(_NCHUNK):
        j = c + _NBUF - 1
        if j < _NCHUNK:
            if c >= 1:
                o[c - 1].wait()
                waited.add(c - 1)
            g[j] = gather(j)
        g[c].wait()
        o[c] = put(c)
    for c in range(_NCHUNK):
        if c not in waited:
            o[c].wait()


@jax.jit
def _gather(table, idx_2d):
    mesh = plsc.VectorSubcoreMesh(core_axis_name="c", subcore_axis_name="s")
    k = functools.partial(
        pl.kernel,
        mesh=mesh,
        out_type=jax.ShapeDtypeStruct((_N, _D), jnp.float32),
        scratch_types=(
            [pltpu.VMEM((_BPW,), jnp.int32)]
            + [pltpu.VMEM((_CH, _D), jnp.float32)] * _NBUF
            + [pltpu.SemaphoreType.DMA] * (2 * _NBUF)
        ),
    )(_gather_body)
    return k(table, idx_2d)


def kernel(input_ids, word_embeddings):
    idx_2d = input_ids.astype(jnp.int32)
    out = _gather(word_embeddings, idx_2d)
    return out.reshape(_B, _S, _D)


# speedup vs baseline: 1.0016x; 1.0016x over previous
import functools

import jax
import jax.numpy as jnp
from jax import lax
from jax.experimental import pallas as pl
from jax.experimental.pallas import tpu as pltpu
from jax.experimental.pallas import tpu_sc as plsc

_VOCAB = 50277
_D = 1024
_B = 4
_S = 4096
_N = _B * _S

_NC = 2
_NS = 16
_NW = _NC * _NS
_BPW = _N // _NW
_CH = 16
_NCHUNK = _BPW // _CH
_NBUF = 6


_IDX0 = 128


def _gather_body(table_hbm, idx_hbm, out_hbm, idx_v, *scr):
    wid = lax.axis_index("s") * _NC + lax.axis_index("c")
    base = wid * _BPW
    bufs = scr[:_NBUF]
    gsems = scr[_NBUF:2 * _NBUF]
    osems = scr[2 * _NBUF:3 * _NBUF]
    isem = scr[3 * _NBUF]
    pltpu.sync_copy(idx_hbm.at[pl.ds(base, _IDX0)], idx_v.at[pl.ds(0, _IDX0)])
    idx_rest = pltpu.async_copy(
        idx_hbm.at[pl.ds(base + _IDX0, _BPW - _IDX0)],
        idx_v.at[pl.ds(_IDX0, _BPW - _IDX0)], isem)

    def gather(c):
        b = c % _NBUF
        return pltpu.async_copy(
            table_hbm.at[idx_v.at[pl.ds(c * _CH, _CH)]], bufs[b], gsems[b])

    def put(c):
        b = c % _NBUF
        return pltpu.async_copy(
            bufs[b], out_hbm.at[pl.ds(base + c * _CH, _CH)], osems[b])

    g = [None] * _NCHUNK
    o = [None] * _NCHUNK
    waited = set()
    idx_waited = False

    def ensure_idx(c):
        nonlocal idx_waited
        if not idx_waited and (c + 1) * _CH > _IDX0:
            idx_rest.wait()
            idx_waited = True

    for c in range(_NBUF - 1):
        ensure_idx(c)
        g[c] = gather(c)
    for c in range(_NCHUNK):
        j = c + _NBUF - 1
        if j < _NCHUNK:
            if c >= 1:
                o[c - 1].wait()
                waited.add(c - 1)
            ensure_idx(j)
            g[j] = gather(j)
        g[c].wait()
        o[c] = put(c)
    for c in range(_NCHUNK):
        if c not in waited:
            o[c].wait()


@jax.jit
def _gather(table, idx_flat):
    mesh = plsc.VectorSubcoreMesh(core_axis_name="c", subcore_axis_name="s")
    k = functools.partial(
        pl.kernel,
        mesh=mesh,
        out_type=jax.ShapeDtypeStruct((_N, _D), jnp.float32),
        scratch_types=(
            [pltpu.VMEM((_BPW,), jnp.int32)]
            + [pltpu.VMEM((_CH, _D), jnp.float32)] * _NBUF
            + [pltpu.SemaphoreType.DMA] * (2 * _NBUF + 1)
        ),
    )(_gather_body)
    return k(table, idx_flat)


def kernel(input_ids, word_embeddings):
    idx_flat = input_ids.reshape(-1).astype(jnp.int32)
    out = _gather(word_embeddings, idx_flat)
    return out.reshape(_B, _S, _D)
